# Initial kernel scaffold; baseline (speedup 1.0000x reference)
import jax
import jax.numpy as jnp
from jax.experimental import pallas as pl


def kernel(reg_preds, cls_preds, anchors, y):
    raise NotImplementedError("write your pallas kernel here")


# speedup vs baseline: 1.9933x; 1.9933x over previous
import functools

import jax
import jax.numpy as jnp
from jax.experimental import pallas as pl
from jax.experimental.pallas import tpu as pltpu

_N = 20000
_TOPK = 200
_G = 100
_ROWS = 160
_LANES = 128
_PADN = _ROWS * _LANES
_HI_KEY = 0x3F800000


def _body(l0, l1, r0, r1, r2, r3, a0, a1, a2, a3, y_sm, out_ref):
    f32 = jnp.float32
    HIGH = jax.lax.Precision.HIGHEST

    l0v = l0[...]
    l1v = l1[...]

    m = jnp.maximum(l0v, l1v)
    e0 = jnp.exp(l0v - m)
    e1 = jnp.exp(l1v - m)
    ssum = e0 + e1
    p0 = e0 / ssum
    p1 = e1 / ssum
    fg = p1 > p0
    s = jnp.where(fg, jnp.maximum(p0, p1), jnp.float32(-1.0))

    ridx = jax.lax.broadcasted_iota(jnp.int32, (_ROWS, _LANES), 0)
    lidx = jax.lax.broadcasted_iota(jnp.int32, (_ROWS, _LANES), 1)
    flat = ridx * _LANES + lidx
    valid = flat < _N
    s = jnp.where(valid, s, jnp.float32(-2.0))

    key = jnp.where(
        s > 0,
        jax.lax.bitcast_convert_type(s, jnp.int32),
        jnp.where(s == -1.0, jnp.int32(-1), jnp.int32(-2)),
    )

    def bs_step(_, lohi):
        lo, hi = lohi
        mid = lo + (hi - lo) // 2
        cnt = jnp.sum((key >= mid).astype(jnp.int32))
        big = cnt >= _TOPK
        return (jnp.where(big, mid, lo), jnp.where(big, hi, mid))

    lo, hi = jax.lax.fori_loop(
        0, 31, bs_step, (jnp.int32(-1), jnp.int32(_HI_KEY))
    )
    v_key = lo

    gt = key > v_key
    tie = key == v_key
    cnt_gt = jnp.sum(gt.astype(jnp.int32))
    needed = _TOPK - cnt_gt

    t_f = tie.astype(f32)
    iu0 = jax.lax.broadcasted_iota(jnp.int32, (_LANES, _LANES), 0)
    iu1 = jax.lax.broadcasted_iota(jnp.int32, (_LANES, _LANES), 1)
    upper = (iu0 <= iu1).astype(f32)
    il0 = jax.lax.broadcasted_iota(jnp.int32, (_ROWS, _ROWS), 0)
    il1 = jax.lax.broadcasted_iota(jnp.int32, (_ROWS, _ROWS), 1)
    lower = (il0 > il1).astype(f32)
    ones_l = jnp.ones((_LANES, _LANES), f32)
    rowpart = jax.lax.dot(lower, t_f, precision=HIGH)
    rowoff = jax.lax.dot(rowpart, ones_l, precision=HIGH)
    intrarow = jax.lax.dot(t_f, upper, precision=HIGH)
    prefix = rowoff + intrarow
    sel = gt | (tie & (prefix <= needed.astype(f32)))

    x = r0[...] * a2[...] + a0[...]
    yy = r1[...] * a3[...] + a1[...]
    w = jnp.exp(r2[...]) * a2[...]
    h = jnp.exp(r3[...]) * a3[...]
    xe = x + w
    ye = yy + h
    area = w * h

    def iou_step(g, td):
        gx = y_sm[g, 0]
        gy = y_sm[g, 1]
        gw = y_sm[g, 2]
        gh = y_sm[g, 3]
        dw = jnp.minimum(xe, gx + gw) - jnp.maximum(x, gx)
        dh = jnp.minimum(ye, gy + gh) - jnp.maximum(yy, gy)
        inter = dw * dh
        iou = inter / (area + gw * gh - inter)
        return td | (iou > 0.3).astype(jnp.int32)

    td = jax.lax.fori_loop(
        0, _G, iou_step, jnp.zeros((_ROWS, _LANES), jnp.int32)
    ) != 0

    term = jnp.where(td, jnp.log(1.0 - s), jnp.log(s))
    loss = jnp.sum(jnp.where(sel, term, jnp.float32(0.0)))
    out_ref[0, 0] = loss


def kernel(reg_preds, cls_preds, anchors, y):
    pad = _PADN - _N

    def col(arr, c):
        return jnp.pad(arr[:, c], (0, pad)).reshape(_ROWS, _LANES)

    ins = (
        col(cls_preds, 0), col(cls_preds, 1),
        col(reg_preds, 0), col(reg_preds, 1), col(reg_preds, 2), col(reg_preds, 3),
        col(anchors, 0), col(anchors, 1), col(anchors, 2), col(anchors, 3),
        y,
    )
    vspec = pl.BlockSpec(memory_space=pltpu.VMEM)
    sspec = pl.BlockSpec(memory_space=pltpu.SMEM)
    out = pl.pallas_call(
        _body,
        out_shape=jax.ShapeDtypeStruct((1, 1), jnp.float32),
        in_specs=[vspec] * 10 + [sspec],
        out_specs=pl.BlockSpec(memory_space=pltpu.SMEM),
    )(*ins)
    return out[0, 0]


# speedup vs baseline: 2.1502x; 1.0787x over previous
import functools

import jax
import jax.numpy as jnp
from jax.experimental import pallas as pl
from jax.experimental.pallas import tpu as pltpu

_N = 20000
_TOPK = 200
_G = 100
_ROWS = 160
_LANES = 128
_PADN = _ROWS * _LANES
_HI_KEY = 0x3F800000
THR = 0.3


def _body(l0, l1, r0, r1, r2, r3, a0, a1, a2, a3, y_sm, out_ref):
    f32 = jnp.float32
    HIGH = jax.lax.Precision.HIGHEST

    l0v = l0[...]
    l1v = l1[...]

    m = jnp.maximum(l0v, l1v)
    e0 = jnp.exp(l0v - m)
    e1 = jnp.exp(l1v - m)
    ssum = e0 + e1
    p0 = e0 / ssum
    p1 = e1 / ssum
    fg = p1 > p0
    s = jnp.where(fg, jnp.maximum(p0, p1), jnp.float32(-1.0))

    ridx = jax.lax.broadcasted_iota(jnp.int32, (_ROWS, _LANES), 0)
    lidx = jax.lax.broadcasted_iota(jnp.int32, (_ROWS, _LANES), 1)
    flat = ridx * _LANES + lidx
    valid = flat < _N
    s = jnp.where(valid, s, jnp.float32(-2.0))

    key = jnp.where(
        s > 0,
        jax.lax.bitcast_convert_type(s, jnp.int32),
        jnp.where(s == -1.0, jnp.int32(-1), jnp.int32(-2)),
    )

    def bs_step(_, lohi):
        lo, hi = lohi
        mid = lo + (hi - lo) // 2
        cnt = jnp.sum((key >= mid).astype(jnp.int32))
        big = cnt >= _TOPK
        return (jnp.where(big, mid, lo), jnp.where(big, hi, mid))

    cnt_pos = jnp.sum((key >= 0x3F000000).astype(jnp.int32))
    lo, hi = jax.lax.fori_loop(
        0, 24, bs_step, (jnp.int32(0x3F000000 - 1), jnp.int32(_HI_KEY))
    )
    v_key = jnp.where(cnt_pos >= _TOPK, lo, jnp.int32(-1))

    gt = key > v_key
    tie = key == v_key
    cnt_gt = jnp.sum(gt.astype(jnp.int32))
    needed = _TOPK - cnt_gt

    t_f = tie.astype(f32)
    iu0 = jax.lax.broadcasted_iota(jnp.int32, (_LANES, _LANES), 0)
    iu1 = jax.lax.broadcasted_iota(jnp.int32, (_LANES, _LANES), 1)
    upper = (iu0 <= iu1).astype(f32)
    il0 = jax.lax.broadcasted_iota(jnp.int32, (_ROWS, _ROWS), 0)
    il1 = jax.lax.broadcasted_iota(jnp.int32, (_ROWS, _ROWS), 1)
    lower = (il0 > il1).astype(f32)
    ones_l = jnp.ones((_LANES, _LANES), f32)
    rowpart = jax.lax.dot(lower, t_f, precision=HIGH)
    rowoff = jax.lax.dot(rowpart, ones_l, precision=HIGH)
    intrarow = jax.lax.dot(t_f, upper, precision=HIGH)
    prefix = rowoff + intrarow
    sel = gt | (tie & (prefix <= needed.astype(f32)))

    x = r0[...] * a2[...] + a0[...]
    yy = r1[...] * a3[...] + a1[...]
    w = jnp.exp(r2[...]) * a2[...]
    h = jnp.exp(r3[...]) * a3[...]
    xe = x + w
    ye = yy + h
    area = w * h

    def iou_step(g, td):
        gx = y_sm[g, 0]
        gy = y_sm[g, 1]
        gw = y_sm[g, 2]
        gh = y_sm[g, 3]
        dw = jnp.minimum(xe, gx + gw) - jnp.maximum(x, gx)
        dh = jnp.minimum(ye, gy + gh) - jnp.maximum(yy, gy)
        inter = dw * dh
        u = (area + gw * gh) - inter
        hit = (inter > jnp.float32(THR) * u) ^ (u < 0)
        return td | hit.astype(jnp.int32)

    td = jax.lax.fori_loop(
        0, _G, iou_step, jnp.zeros((_ROWS, _LANES), jnp.int32),
        unroll=10,
    ) != 0

    term = jnp.where(td, jnp.log(1.0 - s), jnp.log(s))
    loss = jnp.sum(jnp.where(sel, term, jnp.float32(0.0)))
    out_ref[0, 0] = loss


def kernel(reg_preds, cls_preds, anchors, y):
    pad = _PADN - _N

    def col(arr, c):
        return jnp.pad(arr[:, c], (0, pad)).reshape(_ROWS, _LANES)

    ins = (
        col(cls_preds, 0), col(cls_preds, 1),
        col(reg_preds, 0), col(reg_preds, 1), col(reg_preds, 2), col(reg_preds, 3),
        col(anchors, 0), col(anchors, 1), col(anchors, 2), col(anchors, 3),
        y,
    )
    vspec = pl.BlockSpec(memory_space=pltpu.VMEM)
    sspec = pl.BlockSpec(memory_space=pltpu.SMEM)
    out = pl.pallas_call(
        _body,
        out_shape=jax.ShapeDtypeStruct((1, 1), jnp.float32),
        in_specs=[vspec] * 10 + [sspec],
        out_specs=pl.BlockSpec(memory_space=pltpu.SMEM),
    )(*ins)
    return out[0, 0]
